# baseline (speedup 1.0000x reference)
import functools

import jax
import jax.numpy as jnp
from jax import lax
from jax.experimental import pallas as pl
from jax.experimental.pallas import tpu as pltpu
from jax.experimental.pallas import tpu_sc as plsc

N = 10000
E = 320000
D = 128
G = 64
OUTD = 32

NC = 2
NS = 16
NW = NC * NS
CH = 80
EPT = E // NW
NROW = E // CH
RPT = NROW // NW
RSTAGE = 25
NSTAGE = RPT // RSTAGE
NPAIR = 12
RPS = N // NS
ZR = 25
BN = 2000
NB = N // BN
DSTG = 2000

_mesh = plsc.VectorSubcoreMesh(
    core_axis_name="c", subcore_axis_name="s", num_cores=NC, num_subcores=NS)
_sc_params = pltpu.CompilerParams(
    use_tc_tiling_on_sc=False, needs_layout_passes=False)


def _deg_body(col_hbm, ew_hbm, pdeg_hbm, cbuf, wbuf, deg_v):
    cid = lax.axis_index("c")
    sid = lax.axis_index("s")
    wid = cid * NS + sid

    def zero(i, carry):
        deg_v[pl.ds(i * 16, 16)] = jnp.zeros((16,), jnp.float32)
        return carry

    lax.fori_loop(0, N // 16, zero, 0, unroll=8)

    for st in range(EPT // DSTG):
        b0 = wid * EPT + st * DSTG
        pltpu.sync_copy(col_hbm.at[pl.ds(b0, DSTG)], cbuf)
        pltpu.sync_copy(ew_hbm.at[pl.ds(b0, DSTG)], wbuf)

        def body(g, carry):
            c = cbuf[pl.ds(g * 16, 16)]
            w = wbuf[pl.ds(g * 16, 16)]
            plsc.addupdate_scatter(deg_v, [c], w)
            return carry

        lax.fori_loop(0, DSTG // 16, body, 0, unroll=4)

    for kb in range(NB):
        pltpu.sync_copy(deg_v.at[pl.ds(kb * BN, BN)], pdeg_hbm.at[kb, wid])


_deg_kernel = functools.partial(
    pl.kernel,
    out_type=jax.ShapeDtypeStruct((NB, NW, BN), jnp.float32),
    mesh=_mesh,
    compiler_params=_sc_params,
    scratch_types=[
        pltpu.VMEM((DSTG,), jnp.int32),
        pltpu.VMEM((DSTG,), jnp.float32),
        pltpu.VMEM((N,), jnp.float32),
    ],
)(_deg_body)


def _scale(gbuf, sbuf, wbuf, r):
    def ebody(e, carry):
        w = plsc.load_gather(
            wbuf,
            [jnp.full((16,), r, jnp.int32), jnp.full((16,), e, jnp.int32)])
        for q in range(D // 16):
            sl = pl.ds(q * 16, 16)
            sbuf[e, sl] = gbuf[e, sl] * w
        return carry

    lax.fori_loop(0, CH, ebody, 0, unroll=5)


def _edge_body(y_hbm, row_hbm, col_hbm, ew_hbm, out_hbm,
               rbuf, cbuf, wbuf, gA, gB, sA, sB, acc,
               gsemA, gsemB, ssemA, ssemB):
    cid = lax.axis_index("c")
    sid = lax.axis_index("s")
    wid = cid * NS + sid

    def zero(i, carry):
        sA[i // 8, pl.ds((i % 8) * 16, 16)] = jnp.zeros((16,), jnp.float32)
        return carry

    lax.fori_loop(0, ZR * 8, zero, 0, unroll=8)

    def zcopy(k, carry):
        pltpu.sync_copy(sA.at[pl.ds(0, ZR)],
                        acc.at[pl.ds(sid * RPS + k * ZR, ZR)])
        return carry

    lax.fori_loop(0, RPS // ZR, zcopy, 0)
    plsc.subcore_barrier()

    for st in range(NSTAGE):
        r0 = wid * RPT + st * RSTAGE
        pltpu.sync_copy(row_hbm.at[pl.ds(r0, RSTAGE)], rbuf)
        pltpu.sync_copy(col_hbm.at[pl.ds(r0, RSTAGE)], cbuf)
        pltpu.sync_copy(ew_hbm.at[pl.ds(r0, RSTAGE)], wbuf)
        pltpu.async_copy(y_hbm.at[rbuf.at[0]], gA, gsemA)

        def pair(j, carry):
            c0 = 2 * j
            c1 = 2 * j + 1
            pltpu.async_copy(y_hbm.at[rbuf.at[c1]], gB, gsemB)
            pltpu.make_async_copy(y_hbm.at[rbuf.at[c0]], gA, gsemA).wait()

            @pl.when(j > 0)
            def _():
                pltpu.make_async_copy(
                    sA, acc.at[cbuf.at[c0 - 2]], ssemA).wait()

            _scale(gA, sA, wbuf, c0)
            pltpu.async_copy(sA, acc.at[cbuf.at[c0]], ssemA, add=True)
            pltpu.async_copy(y_hbm.at[rbuf.at[c0 + 2]], gA, gsemA)
            pltpu.make_async_copy(y_hbm.at[rbuf.at[c1]], gB, gsemB).wait()

            @pl.when(j > 0)
            def _():
                pltpu.make_async_copy(
                    sB, acc.at[cbuf.at[c1 - 2]], ssemB).wait()

            _scale(gB, sB, wbuf, c1)
            pltpu.async_copy(sB, acc.at[cbuf.at[c1]], ssemB, add=True)
            return carry

        lax.fori_loop(0, NPAIR, pair, 0)
        tc = RSTAGE - 1
        pltpu.make_async_copy(y_hbm.at[rbuf.at[tc]], gA, gsemA).wait()
        pltpu.make_async_copy(sA, acc.at[cbuf.at[tc - 2]], ssemA).wait()
        _scale(gA, sA, wbuf, tc)
        pltpu.async_copy(sA, acc.at[cbuf.at[tc]], ssemA, add=True)
        pltpu.make_async_copy(sB, acc.at[cbuf.at[tc - 1]], ssemB).wait()
        pltpu.make_async_copy(sA, acc.at[cbuf.at[tc]], ssemA).wait()

    plsc.subcore_barrier()

    def wcopy(k, carry):
        r0 = sid * RPS + k * ZR
        pltpu.sync_copy(acc.at[pl.ds(r0, ZR)], sA.at[pl.ds(0, ZR)])
        pltpu.sync_copy(sA.at[pl.ds(0, ZR)], out_hbm.at[cid, pl.ds(r0, ZR)])
        return carry

    lax.fori_loop(0, RPS // ZR, wcopy, 0)


_edge_kernel = functools.partial(
    pl.kernel,
    out_type=jax.ShapeDtypeStruct((NC, N, D), jnp.float32),
    mesh=_mesh,
    compiler_params=_sc_params,
    scratch_types=[
        pltpu.VMEM((RSTAGE, CH), jnp.int32),
        pltpu.VMEM((RSTAGE, CH), jnp.int32),
        pltpu.VMEM((RSTAGE, CH), jnp.float32),
        pltpu.VMEM((CH, D), jnp.float32),
        pltpu.VMEM((CH, D), jnp.float32),
        pltpu.VMEM((CH, D), jnp.float32),
        pltpu.VMEM((CH, D), jnp.float32),
        pltpu.VMEM_SHARED((N, D), jnp.float32),
        pltpu.SemaphoreType.DMA,
        pltpu.SemaphoreType.DMA,
        pltpu.SemaphoreType.DMA,
        pltpu.SemaphoreType.DMA,
    ],
)(_edge_body)


def _dinv_block(pdeg_ref):
    deg = 1.0 + jnp.sum(pdeg_ref[0], axis=0)
    return lax.rsqrt(jnp.maximum(deg, 1e-12))


def _prep_body(x_ref, w_ref, pdeg_ref, y_ref):
    dinv = _dinv_block(pdeg_ref)
    xw = jnp.dot(x_ref[...], w_ref[...], preferred_element_type=jnp.float32)
    y_ref[...] = dinv[:, None] * xw


def _tc_prep(x, Wc1, pdeg):
    return pl.pallas_call(
        _prep_body,
        grid=(NB,),
        in_specs=[
            pl.BlockSpec((BN, D), lambda i: (i, 0)),
            pl.BlockSpec((D, D), lambda i: (0, 0)),
            pl.BlockSpec((1, NW, BN), lambda i: (i, 0, 0)),
        ],
        out_specs=pl.BlockSpec((BN, D), lambda i: (i, 0)),
        out_shape=jax.ShapeDtypeStruct((N, D), jnp.float32),
    )(x, Wc1, pdeg)


def _mid_body(acc_ref, y_ref, pdeg_ref, b_ref, w_ref, y2_ref):
    dinv = _dinv_block(pdeg_ref)
    t = acc_ref[0] + acc_ref[1] + y_ref[...]
    h = jnp.maximum(dinv[:, None] * t + b_ref[...], 0.0)
    hw = jnp.dot(h, w_ref[...], preferred_element_type=jnp.float32)
    y2_ref[...] = dinv[:, None] * hw


def _tc_mid(acc, y, pdeg, b, W):
    return pl.pallas_call(
        _mid_body,
        grid=(NB,),
        in_specs=[
            pl.BlockSpec((NC, BN, D), lambda i: (0, i, 0)),
            pl.BlockSpec((BN, D), lambda i: (i, 0)),
            pl.BlockSpec((1, NW, BN), lambda i: (i, 0, 0)),
            pl.BlockSpec((1, D), lambda i: (0, 0)),
            pl.BlockSpec((D, D), lambda i: (0, 0)),
        ],
        out_specs=pl.BlockSpec((BN, D), lambda i: (i, 0)),
        out_shape=jax.ShapeDtypeStruct((N, D), jnp.float32),
    )(acc, y, pdeg, b.reshape(1, D), W)


def _final_body(acc_ref, y_ref, pdeg_ref, b_ref, batch_ref,
                w0_ref, b0_ref, w1_ref, b1_ref, w2_ref, b2_ref,
                out_ref, sums, cnts):
    i = pl.program_id(0)
    dinv = _dinv_block(pdeg_ref)
    t = acc_ref[0] + acc_ref[1] + y_ref[...]
    h = jnp.maximum(dinv[:, None] * t + b_ref[...], 0.0)

    seg = lax.broadcasted_iota(jnp.int32, (G, BN), 0)
    oh = (batch_ref[0, 0, :][None, :] == seg).astype(jnp.float32)

    @pl.when(i == 0)
    def _():
        sums[...] = jnp.zeros((G, D), jnp.float32)
        cnts[...] = jnp.zeros((G, D), jnp.float32)

    sums[...] += jnp.dot(oh, h, preferred_element_type=jnp.float32)
    cnts[...] += jnp.dot(oh, jnp.ones((BN, D), jnp.float32),
                         preferred_element_type=jnp.float32)

    @pl.when(i == NB - 1)
    def _():
        g = sums[...] / jnp.maximum(cnts[...], 1.0)
        g = jnp.maximum(
            jnp.dot(g, w0_ref[...], preferred_element_type=jnp.float32)
            + b0_ref[...], 0.0)
        g = jnp.maximum(
            jnp.dot(g, w1_ref[...], preferred_element_type=jnp.float32)
            + b1_ref[...], 0.0)
        out_ref[...] = (
            jnp.dot(g, w2_ref[...], preferred_element_type=jnp.float32)
            + b2_ref[...])


def _tc_final(acc, y, pdeg, b, batch3, Wl0, bl0, Wl1, bl1, Wl2, bl2):
    return pl.pallas_call(
        _final_body,
        grid=(NB,),
        in_specs=[
            pl.BlockSpec((NC, BN, D), lambda i: (0, i, 0)),
            pl.BlockSpec((BN, D), lambda i: (i, 0)),
            pl.BlockSpec((1, NW, BN), lambda i: (i, 0, 0)),
            pl.BlockSpec((1, D), lambda i: (0, 0)),
            pl.BlockSpec((1, 1, BN), lambda i: (i, 0, 0)),
            pl.BlockSpec((D, D), lambda i: (0, 0)),
            pl.BlockSpec((1, D), lambda i: (0, 0)),
            pl.BlockSpec((D, D), lambda i: (0, 0)),
            pl.BlockSpec((1, D), lambda i: (0, 0)),
            pl.BlockSpec((D, OUTD), lambda i: (0, 0)),
            pl.BlockSpec((1, OUTD), lambda i: (0, 0)),
        ],
        out_specs=pl.BlockSpec((G, OUTD), lambda i: (0, 0)),
        out_shape=jax.ShapeDtypeStruct((G, OUTD), jnp.float32),
        scratch_shapes=[
            pltpu.VMEM((G, D), jnp.float32),
            pltpu.VMEM((G, D), jnp.float32),
        ],
    )(acc, y, pdeg, b.reshape(1, D), batch3,
      Wl0, bl0.reshape(1, D), Wl1, bl1.reshape(1, D),
      Wl2, bl2.reshape(1, OUTD))


@jax.jit
def kernel(x, edge_attr, edge_index, batch,
           Wc1, bc1, Wc2, bc2, Wl0, bl0, Wl1, bl1, Wl2, bl2):
    row = edge_index[0].astype(jnp.int32)
    col = edge_index[1].astype(jnp.int32)
    row2 = row.reshape(NROW, CH)
    col2 = col.reshape(NROW, CH)
    ew2 = edge_attr.reshape(NROW, CH)
    batch3 = batch.astype(jnp.int32).reshape(NB, 1, BN)

    pdeg = _deg_kernel(col, edge_attr)
    y1 = _tc_prep(x, Wc1, pdeg)
    acc1 = _edge_kernel(y1, row2, col2, ew2)
    y2 = _tc_mid(acc1, y1, pdeg, bc1, Wc2)
    acc2 = _edge_kernel(y2, row2, col2, ew2)
    return _tc_final(acc2, y2, pdeg, bc2, batch3,
                     Wl0, bl0, Wl1, bl1, Wl2, bl2)


# speedup vs baseline: 2.0968x; 2.0968x over previous
import functools

import jax
import jax.numpy as jnp
from jax import lax
from jax.experimental import pallas as pl
from jax.experimental.pallas import tpu as pltpu
from jax.experimental.pallas import tpu_sc as plsc

N = 10000
E = 320000
D = 128
G = 64
OUTD = 32

NC = 2
NS = 16
NW = NC * NS
CH = 125
EPT = E // NW
NROW = E // CH
RPT = NROW // NW
RSTAGE = 16
NSTAGE = RPT // RSTAGE
NPAIR = RSTAGE // 2
RPS = N // NS
ZR = 125
BN = 2000
NB = N // BN
DSTG = 2000

_mesh = plsc.VectorSubcoreMesh(
    core_axis_name="c", subcore_axis_name="s", num_cores=NC, num_subcores=NS)
_sc_params = pltpu.CompilerParams(
    use_tc_tiling_on_sc=False, needs_layout_passes=False)


def _deg_body(col_hbm, ew_hbm, pdeg_hbm, cbuf, wbuf, deg_v):
    cid = lax.axis_index("c")
    sid = lax.axis_index("s")
    wid = cid * NS + sid

    def zero(i, carry):
        deg_v[pl.ds(i * 16, 16)] = jnp.zeros((16,), jnp.float32)
        return carry

    lax.fori_loop(0, N // 16, zero, 0, unroll=8)

    for st in range(EPT // DSTG):
        b0 = wid * EPT + st * DSTG
        pltpu.sync_copy(col_hbm.at[pl.ds(b0, DSTG)], cbuf)
        pltpu.sync_copy(ew_hbm.at[pl.ds(b0, DSTG)], wbuf)

        def body(g, carry):
            c = cbuf[pl.ds(g * 16, 16)]
            w = wbuf[pl.ds(g * 16, 16)]
            plsc.addupdate_scatter(deg_v, [c], w)
            return carry

        lax.fori_loop(0, DSTG // 16, body, 0, unroll=4)

    for kb in range(NB):
        pltpu.sync_copy(deg_v.at[pl.ds(kb * BN, BN)], pdeg_hbm.at[kb, wid])


_deg_kernel = functools.partial(
    pl.kernel,
    out_type=jax.ShapeDtypeStruct((NB, NW, BN), jnp.float32),
    mesh=_mesh,
    compiler_params=_sc_params,
    scratch_types=[
        pltpu.VMEM((DSTG,), jnp.int32),
        pltpu.VMEM((DSTG,), jnp.float32),
        pltpu.VMEM((N,), jnp.float32),
    ],
)(_deg_body)


def _scale(buf, wbuf, r):
    def ebody(e, carry):
        w = plsc.load_gather(
            wbuf,
            [jnp.full((16,), r, jnp.int32), jnp.full((16,), e, jnp.int32)])
        for q in range(D // 16):
            sl = pl.ds(q * 16, 16)
            buf[e, sl] = buf[e, sl] * w
        return carry

    lax.fori_loop(0, CH, ebody, 0, unroll=5)


def _edge_body(y_hbm, row_hbm, col_hbm, ew_hbm, out_hbm,
               rbuf, cbuf, wbuf, gA, gB, acc, gsemA, gsemB):
    cid = lax.axis_index("c")
    sid = lax.axis_index("s")
    wid = cid * NS + sid

    def zero(i, carry):
        gA[i // 8, pl.ds((i % 8) * 16, 16)] = jnp.zeros((16,), jnp.float32)
        return carry

    lax.fori_loop(0, ZR * 8, zero, 0, unroll=8)
    for k in range(RPS // ZR):
        pltpu.sync_copy(gA, acc.at[pl.ds(sid * RPS + k * ZR, ZR)])
    plsc.subcore_barrier()

    for st in range(NSTAGE):
        r0 = wid * RPT + st * RSTAGE
        pltpu.sync_copy(row_hbm.at[pl.ds(r0, RSTAGE)], rbuf)
        pltpu.sync_copy(col_hbm.at[pl.ds(r0, RSTAGE)], cbuf)
        pltpu.sync_copy(ew_hbm.at[pl.ds(r0, RSTAGE)], wbuf)
        pltpu.async_copy(y_hbm.at[rbuf.at[0]], gA, gsemA)

        def pair(j, carry):
            c0 = 2 * j
            c1 = 2 * j + 1
            pltpu.async_copy(y_hbm.at[rbuf.at[c1]], gB, gsemB)
            pltpu.make_async_copy(y_hbm.at[rbuf.at[c0]], gA, gsemA).wait()
            _scale(gA, wbuf, c0)
            pltpu.sync_copy(gA, acc.at[cbuf.at[c0]], add=True)

            @pl.when(j < NPAIR - 1)
            def _():
                pltpu.async_copy(y_hbm.at[rbuf.at[c0 + 2]], gA, gsemA)

            pltpu.make_async_copy(y_hbm.at[rbuf.at[c1]], gB, gsemB).wait()
            _scale(gB, wbuf, c1)
            pltpu.sync_copy(gB, acc.at[cbuf.at[c1]], add=True)
            return carry

        lax.fori_loop(0, NPAIR, pair, 0)

    plsc.subcore_barrier()
    for k in range(RPS // ZR):
        r0 = sid * RPS + k * ZR
        pltpu.sync_copy(acc.at[pl.ds(r0, ZR)], gA)
        pltpu.sync_copy(gA, out_hbm.at[cid, pl.ds(r0, ZR)])


_edge_kernel = functools.partial(
    pl.kernel,
    out_type=jax.ShapeDtypeStruct((NC, N, D), jnp.float32),
    mesh=_mesh,
    compiler_params=_sc_params,
    scratch_types=[
        pltpu.VMEM((RSTAGE, CH), jnp.int32),
        pltpu.VMEM((RSTAGE, CH), jnp.int32),
        pltpu.VMEM((RSTAGE, CH), jnp.float32),
        pltpu.VMEM((CH, D), jnp.float32),
        pltpu.VMEM((CH, D), jnp.float32),
        pltpu.VMEM_SHARED((N, D), jnp.float32),
        pltpu.SemaphoreType.DMA,
        pltpu.SemaphoreType.DMA,
    ],
)(_edge_body)


def _dinv_block(pdeg_ref):
    deg = 1.0 + jnp.sum(pdeg_ref[0], axis=0)
    return lax.rsqrt(jnp.maximum(deg, 1e-12))


def _prep_body(x_ref, w_ref, pdeg_ref, y_ref):
    dinv = _dinv_block(pdeg_ref)
    xw = jnp.dot(x_ref[...], w_ref[...], preferred_element_type=jnp.float32)
    y_ref[...] = dinv[:, None] * xw


def _tc_prep(x, Wc1, pdeg):
    return pl.pallas_call(
        _prep_body,
        grid=(NB,),
        in_specs=[
            pl.BlockSpec((BN, D), lambda i: (i, 0)),
            pl.BlockSpec((D, D), lambda i: (0, 0)),
            pl.BlockSpec((1, NW, BN), lambda i: (i, 0, 0)),
        ],
        out_specs=pl.BlockSpec((BN, D), lambda i: (i, 0)),
        out_shape=jax.ShapeDtypeStruct((N, D), jnp.float32),
    )(x, Wc1, pdeg)


def _mid_body(acc_ref, y_ref, pdeg_ref, b_ref, w_ref, y2_ref):
    dinv = _dinv_block(pdeg_ref)
    t = acc_ref[0] + acc_ref[1] + y_ref[...]
    h = jnp.maximum(dinv[:, None] * t + b_ref[...], 0.0)
    hw = jnp.dot(h, w_ref[...], preferred_element_type=jnp.float32)
    y2_ref[...] = dinv[:, None] * hw


def _tc_mid(acc, y, pdeg, b, W):
    return pl.pallas_call(
        _mid_body,
        grid=(NB,),
        in_specs=[
            pl.BlockSpec((NC, BN, D), lambda i: (0, i, 0)),
            pl.BlockSpec((BN, D), lambda i: (i, 0)),
            pl.BlockSpec((1, NW, BN), lambda i: (i, 0, 0)),
            pl.BlockSpec((1, D), lambda i: (0, 0)),
            pl.BlockSpec((D, D), lambda i: (0, 0)),
        ],
        out_specs=pl.BlockSpec((BN, D), lambda i: (i, 0)),
        out_shape=jax.ShapeDtypeStruct((N, D), jnp.float32),
    )(acc, y, pdeg, b.reshape(1, D), W)


def _final_body(acc_ref, y_ref, pdeg_ref, b_ref, batch_ref,
                w0_ref, b0_ref, w1_ref, b1_ref, w2_ref, b2_ref,
                out_ref, sums, cnts):
    i = pl.program_id(0)
    dinv = _dinv_block(pdeg_ref)
    t = acc_ref[0] + acc_ref[1] + y_ref[...]
    h = jnp.maximum(dinv[:, None] * t + b_ref[...], 0.0)

    seg = lax.broadcasted_iota(jnp.int32, (G, BN), 0)
    oh = (batch_ref[0, 0, :][None, :] == seg).astype(jnp.float32)

    @pl.when(i == 0)
    def _():
        sums[...] = jnp.zeros((G, D), jnp.float32)
        cnts[...] = jnp.zeros((G, D), jnp.float32)

    sums[...] += jnp.dot(oh, h, preferred_element_type=jnp.float32)
    cnts[...] += jnp.dot(oh, jnp.ones((BN, D), jnp.float32),
                         preferred_element_type=jnp.float32)

    @pl.when(i == NB - 1)
    def _():
        g = sums[...] / jnp.maximum(cnts[...], 1.0)
        g = jnp.maximum(
            jnp.dot(g, w0_ref[...], preferred_element_type=jnp.float32)
            + b0_ref[...], 0.0)
        g = jnp.maximum(
            jnp.dot(g, w1_ref[...], preferred_element_type=jnp.float32)
            + b1_ref[...], 0.0)
        out_ref[...] = (
            jnp.dot(g, w2_ref[...], preferred_element_type=jnp.float32)
            + b2_ref[...])


def _tc_final(acc, y, pdeg, b, batch3, Wl0, bl0, Wl1, bl1, Wl2, bl2):
    return pl.pallas_call(
        _final_body,
        grid=(NB,),
        in_specs=[
            pl.BlockSpec((NC, BN, D), lambda i: (0, i, 0)),
            pl.BlockSpec((BN, D), lambda i: (i, 0)),
            pl.BlockSpec((1, NW, BN), lambda i: (i, 0, 0)),
            pl.BlockSpec((1, D), lambda i: (0, 0)),
            pl.BlockSpec((1, 1, BN), lambda i: (i, 0, 0)),
            pl.BlockSpec((D, D), lambda i: (0, 0)),
            pl.BlockSpec((1, D), lambda i: (0, 0)),
            pl.BlockSpec((D, D), lambda i: (0, 0)),
            pl.BlockSpec((1, D), lambda i: (0, 0)),
            pl.BlockSpec((D, OUTD), lambda i: (0, 0)),
            pl.BlockSpec((1, OUTD), lambda i: (0, 0)),
        ],
        out_specs=pl.BlockSpec((G, OUTD), lambda i: (0, 0)),
        out_shape=jax.ShapeDtypeStruct((G, OUTD), jnp.float32),
        scratch_shapes=[
            pltpu.VMEM((G, D), jnp.float32),
            pltpu.VMEM((G, D), jnp.float32),
        ],
    )(acc, y, pdeg, b.reshape(1, D), batch3,
      Wl0, bl0.reshape(1, D), Wl1, bl1.reshape(1, D),
      Wl2, bl2.reshape(1, OUTD))


@jax.jit
def kernel(x, edge_attr, edge_index, batch,
           Wc1, bc1, Wc2, bc2, Wl0, bl0, Wl1, bl1, Wl2, bl2):
    row = edge_index[0].astype(jnp.int32)
    col = edge_index[1].astype(jnp.int32)
    row2 = row.reshape(NROW, CH)
    col2 = col.reshape(NROW, CH)
    ew2 = edge_attr.reshape(NROW, CH)
    batch3 = batch.astype(jnp.int32).reshape(NB, 1, BN)

    pdeg = _deg_kernel(col, edge_attr)
    y1 = _tc_prep(x, Wc1, pdeg)
    acc1 = _edge_kernel(y1, row2, col2, ew2)
    y2 = _tc_mid(acc1, y1, pdeg, bc1, Wc2)
    acc2 = _edge_kernel(y2, row2, col2, ew2)
    return _tc_final(acc2, y2, pdeg, bc2, batch3,
                     Wl0, bl0, Wl1, bl1, Wl2, bl2)


# speedup vs baseline: 2.2205x; 1.0590x over previous
import functools

import jax
import jax.numpy as jnp
from jax import lax
from jax.experimental import pallas as pl
from jax.experimental.pallas import tpu as pltpu
from jax.experimental.pallas import tpu_sc as plsc

N = 10000
E = 320000
D = 128
G = 64
OUTD = 32

NC = 2
NS = 16
NW = NC * NS
CH = 125
EPT = E // NW
NROW = E // CH
RPT = NROW // NW
RSTAGE = 8
NSTAGE = RPT // RSTAGE
RPS = N // NS
ZR = 125
BN = 2000
NB = N // BN
DSTG = 2000

_mesh = plsc.VectorSubcoreMesh(
    core_axis_name="c", subcore_axis_name="s", num_cores=NC, num_subcores=NS)
_sc_params = pltpu.CompilerParams(
    use_tc_tiling_on_sc=False, needs_layout_passes=False)


def _deg_body(col_hbm, ew_hbm, pdeg_hbm, cbuf, wbuf, deg_v):
    cid = lax.axis_index("c")
    sid = lax.axis_index("s")
    wid = cid * NS + sid

    def zero(i, carry):
        deg_v[pl.ds(i * 16, 16)] = jnp.zeros((16,), jnp.float32)
        return carry

    lax.fori_loop(0, N // 16, zero, 0, unroll=8)

    for st in range(EPT // DSTG):
        b0 = wid * EPT + st * DSTG
        pltpu.sync_copy(col_hbm.at[pl.ds(b0, DSTG)], cbuf)
        pltpu.sync_copy(ew_hbm.at[pl.ds(b0, DSTG)], wbuf)

        def body(g, carry):
            c = cbuf[pl.ds(g * 16, 16)]
            w = wbuf[pl.ds(g * 16, 16)]
            plsc.addupdate_scatter(deg_v, [c], w)
            return carry

        lax.fori_loop(0, DSTG // 16, body, 0, unroll=4)

    for kb in range(NB):
        pltpu.sync_copy(deg_v.at[pl.ds(kb * BN, BN)], pdeg_hbm.at[kb, wid])


_deg_kernel = functools.partial(
    pl.kernel,
    out_type=jax.ShapeDtypeStruct((NB, NW, BN), jnp.float32),
    mesh=_mesh,
    compiler_params=_sc_params,
    scratch_types=[
        pltpu.VMEM((DSTG,), jnp.int32),
        pltpu.VMEM((DSTG,), jnp.float32),
        pltpu.VMEM((N,), jnp.float32),
    ],
)(_deg_body)


def _scale(buf, wbuf, r):
    def ebody(e, carry):
        w = plsc.load_gather(
            wbuf,
            [jnp.full((16,), r, jnp.int32), jnp.full((16,), e, jnp.int32)])
        for q in range(D // 16):
            sl = pl.ds(q * 16, 16)
            buf[e, sl] = buf[e, sl] * w
        return carry

    lax.fori_loop(0, CH, ebody, 0, unroll=5)


def _edge_body(y_hbm, row_hbm, col_hbm, ew_hbm, out_hbm,
               rbuf, cbuf, wbuf, gA, gB, gC, acc,
               gsemA, gsemB, gsemC, ssemA, ssemB, ssemC):
    cid = lax.axis_index("c")
    sid = lax.axis_index("s")
    wid = cid * NS + sid

    def zero(i, carry):
        gA[i // 8, pl.ds((i % 8) * 16, 16)] = jnp.zeros((16,), jnp.float32)
        return carry

    lax.fori_loop(0, ZR * 8, zero, 0, unroll=8)
    for k in range(RPS // ZR):
        pltpu.sync_copy(gA, acc.at[pl.ds(sid * RPS + k * ZR, ZR)])
    plsc.subcore_barrier()

    bufs = (gA, gB, gC)
    gsems = (gsemA, gsemB, gsemC)
    ssems = (ssemA, ssemB, ssemC)

    def issue_g(b, c):
        pltpu.async_copy(y_hbm.at[rbuf.at[c]], bufs[b], gsems[b])

    def wait_g(b, c):
        pltpu.make_async_copy(y_hbm.at[rbuf.at[c]], bufs[b], gsems[b]).wait()

    def issue_s(b, c):
        pltpu.async_copy(bufs[b], acc.at[cbuf.at[c]], ssems[b], add=True)

    def wait_s(b, c):
        pltpu.make_async_copy(bufs[b], acc.at[cbuf.at[c]], ssems[b]).wait()

    def stage(st, carry):
        r0 = wid * RPT + st * RSTAGE
        pltpu.sync_copy(row_hbm.at[pl.ds(r0, RSTAGE)], rbuf)
        pltpu.sync_copy(col_hbm.at[pl.ds(r0, RSTAGE)], cbuf)
        pltpu.sync_copy(ew_hbm.at[pl.ds(r0, RSTAGE)], wbuf)
        issue_g(0, 0)
        issue_g(1, 1)
        wait_g(0, 0); _scale(gA, wbuf, 0); issue_s(0, 0)
        issue_g(2, 2)
        wait_g(1, 1); _scale(gB, wbuf, 1); issue_s(1, 1)
        wait_s(0, 0); issue_g(0, 3)
        wait_g(2, 2); _scale(gC, wbuf, 2); issue_s(2, 2)
        wait_s(1, 1); issue_g(1, 4)
        wait_g(0, 3); _scale(gA, wbuf, 3); issue_s(0, 3)
        wait_s(2, 2); issue_g(2, 5)
        wait_g(1, 4); _scale(gB, wbuf, 4); issue_s(1, 4)
        wait_s(0, 3); issue_g(0, 6)
        wait_g(2, 5); _scale(gC, wbuf, 5); issue_s(2, 5)
        wait_s(1, 4); issue_g(1, 7)
        wait_g(0, 6); _scale(gA, wbuf, 6); issue_s(0, 6)
        wait_s(2, 5)
        wait_g(1, 7); _scale(gB, wbuf, 7); issue_s(1, 7)
        wait_s(0, 6)
        wait_s(1, 7)
        return carry

    lax.fori_loop(0, NSTAGE, stage, 0)

    plsc.subcore_barrier()
    for k in range(RPS // ZR):
        r0 = sid * RPS + k * ZR
        pltpu.sync_copy(acc.at[pl.ds(r0, ZR)], gA)
        pltpu.sync_copy(gA, out_hbm.at[cid, pl.ds(r0, ZR)])


_edge_kernel = functools.partial(
    pl.kernel,
    out_type=jax.ShapeDtypeStruct((NC, N, D), jnp.float32),
    mesh=_mesh,
    compiler_params=_sc_params,
    scratch_types=[
        pltpu.VMEM((RSTAGE, CH), jnp.int32),
        pltpu.VMEM((RSTAGE, CH), jnp.int32),
        pltpu.VMEM((RSTAGE, CH), jnp.float32),
        pltpu.VMEM((CH, D), jnp.float32),
        pltpu.VMEM((CH, D), jnp.float32),
        pltpu.VMEM((CH, D), jnp.float32),
        pltpu.VMEM_SHARED((N, D), jnp.float32),
        pltpu.SemaphoreType.DMA,
        pltpu.SemaphoreType.DMA,
        pltpu.SemaphoreType.DMA,
        pltpu.SemaphoreType.DMA,
        pltpu.SemaphoreType.DMA,
        pltpu.SemaphoreType.DMA,
    ],
)(_edge_body)


def _dinv_block(pdeg_ref):
    deg = 1.0 + jnp.sum(pdeg_ref[0], axis=0)
    return lax.rsqrt(jnp.maximum(deg, 1e-12))


def _prep_body(x_ref, w_ref, pdeg_ref, y_ref):
    dinv = _dinv_block(pdeg_ref)
    xw = jnp.dot(x_ref[...], w_ref[...], preferred_element_type=jnp.float32)
    y_ref[...] = dinv[:, None] * xw


def _tc_prep(x, Wc1, pdeg):
    return pl.pallas_call(
        _prep_body,
        grid=(NB,),
        in_specs=[
            pl.BlockSpec((BN, D), lambda i: (i, 0)),
            pl.BlockSpec((D, D), lambda i: (0, 0)),
            pl.BlockSpec((1, NW, BN), lambda i: (i, 0, 0)),
        ],
        out_specs=pl.BlockSpec((BN, D), lambda i: (i, 0)),
        out_shape=jax.ShapeDtypeStruct((N, D), jnp.float32),
    )(x, Wc1, pdeg)


def _mid_body(acc_ref, y_ref, pdeg_ref, b_ref, w_ref, y2_ref):
    dinv = _dinv_block(pdeg_ref)
    t = acc_ref[0] + acc_ref[1] + y_ref[...]
    h = jnp.maximum(dinv[:, None] * t + b_ref[...], 0.0)
    hw = jnp.dot(h, w_ref[...], preferred_element_type=jnp.float32)
    y2_ref[...] = dinv[:, None] * hw


def _tc_mid(acc, y, pdeg, b, W):
    return pl.pallas_call(
        _mid_body,
        grid=(NB,),
        in_specs=[
            pl.BlockSpec((NC, BN, D), lambda i: (0, i, 0)),
            pl.BlockSpec((BN, D), lambda i: (i, 0)),
            pl.BlockSpec((1, NW, BN), lambda i: (i, 0, 0)),
            pl.BlockSpec((1, D), lambda i: (0, 0)),
            pl.BlockSpec((D, D), lambda i: (0, 0)),
        ],
        out_specs=pl.BlockSpec((BN, D), lambda i: (i, 0)),
        out_shape=jax.ShapeDtypeStruct((N, D), jnp.float32),
    )(acc, y, pdeg, b.reshape(1, D), W)


def _final_body(acc_ref, y_ref, pdeg_ref, b_ref, batch_ref,
                w0_ref, b0_ref, w1_ref, b1_ref, w2_ref, b2_ref,
                out_ref, sums, cnts):
    i = pl.program_id(0)
    dinv = _dinv_block(pdeg_ref)
    t = acc_ref[0] + acc_ref[1] + y_ref[...]
    h = jnp.maximum(dinv[:, None] * t + b_ref[...], 0.0)

    seg = lax.broadcasted_iota(jnp.int32, (G, BN), 0)
    oh = (batch_ref[0, 0, :][None, :] == seg).astype(jnp.float32)

    @pl.when(i == 0)
    def _():
        sums[...] = jnp.zeros((G, D), jnp.float32)
        cnts[...] = jnp.zeros((G, D), jnp.float32)

    sums[...] += jnp.dot(oh, h, preferred_element_type=jnp.float32)
    cnts[...] += jnp.dot(oh, jnp.ones((BN, D), jnp.float32),
                         preferred_element_type=jnp.float32)

    @pl.when(i == NB - 1)
    def _():
        g = sums[...] / jnp.maximum(cnts[...], 1.0)
        g = jnp.maximum(
            jnp.dot(g, w0_ref[...], preferred_element_type=jnp.float32)
            + b0_ref[...], 0.0)
        g = jnp.maximum(
            jnp.dot(g, w1_ref[...], preferred_element_type=jnp.float32)
            + b1_ref[...], 0.0)
        out_ref[...] = (
            jnp.dot(g, w2_ref[...], preferred_element_type=jnp.float32)
            + b2_ref[...])


def _tc_final(acc, y, pdeg, b, batch3, Wl0, bl0, Wl1, bl1, Wl2, bl2):
    return pl.pallas_call(
        _final_body,
        grid=(NB,),
        in_specs=[
            pl.BlockSpec((NC, BN, D), lambda i: (0, i, 0)),
            pl.BlockSpec((BN, D), lambda i: (i, 0)),
            pl.BlockSpec((1, NW, BN), lambda i: (i, 0, 0)),
            pl.BlockSpec((1, D), lambda i: (0, 0)),
            pl.BlockSpec((1, 1, BN), lambda i: (i, 0, 0)),
            pl.BlockSpec((D, D), lambda i: (0, 0)),
            pl.BlockSpec((1, D), lambda i: (0, 0)),
            pl.BlockSpec((D, D), lambda i: (0, 0)),
            pl.BlockSpec((1, D), lambda i: (0, 0)),
            pl.BlockSpec((D, OUTD), lambda i: (0, 0)),
            pl.BlockSpec((1, OUTD), lambda i: (0, 0)),
        ],
        out_specs=pl.BlockSpec((G, OUTD), lambda i: (0, 0)),
        out_shape=jax.ShapeDtypeStruct((G, OUTD), jnp.float32),
        scratch_shapes=[
            pltpu.VMEM((G, D), jnp.float32),
            pltpu.VMEM((G, D), jnp.float32),
        ],
    )(acc, y, pdeg, b.reshape(1, D), batch3,
      Wl0, bl0.reshape(1, D), Wl1, bl1.reshape(1, D),
      Wl2, bl2.reshape(1, OUTD))


@jax.jit
def kernel(x, edge_attr, edge_index, batch,
           Wc1, bc1, Wc2, bc2, Wl0, bl0, Wl1, bl1, Wl2, bl2):
    row = edge_index[0].astype(jnp.int32)
    col = edge_index[1].astype(jnp.int32)
    row2 = row.reshape(NROW, CH)
    col2 = col.reshape(NROW, CH)
    ew2 = edge_attr.reshape(NROW, CH)
    batch3 = batch.astype(jnp.int32).reshape(NB, 1, BN)

    pdeg = _deg_kernel(col, edge_attr)
    y1 = _tc_prep(x, Wc1, pdeg)
    acc1 = _edge_kernel(y1, row2, col2, ew2)
    y2 = _tc_mid(acc1, y1, pdeg, bc1, Wc2)
    acc2 = _edge_kernel(y2, row2, col2, ew2)
    return _tc_final(acc2, y2, pdeg, bc2, batch3,
                     Wl0, bl0, Wl1, bl1, Wl2, bl2)


# speedup vs baseline: 2.3052x; 1.0381x over previous
import functools

import jax
import jax.numpy as jnp
from jax import lax
from jax.experimental import pallas as pl
from jax.experimental.pallas import tpu as pltpu
from jax.experimental.pallas import tpu_sc as plsc

N = 10000
E = 320000
D = 128
G = 64
OUTD = 32

NC = 2
NS = 16
NW = NC * NS
CH = 125
EPT = E // NW
NROW = E // CH
RPT = NROW // NW
RSTAGE = 8
NSTAGE = RPT // RSTAGE
RPS = N // NS
ZR = 125
BN = 2000
NB = N // BN
DSTG = 2000

_mesh = plsc.VectorSubcoreMesh(
    core_axis_name="c", subcore_axis_name="s", num_cores=NC, num_subcores=NS)
_sc_params = pltpu.CompilerParams(
    use_tc_tiling_on_sc=False, needs_layout_passes=False)


def _deg_body(col_hbm, ew_hbm, pdeg_hbm, cbuf, wbuf, deg_v):
    cid = lax.axis_index("c")
    sid = lax.axis_index("s")
    wid = cid * NS + sid

    def zero(i, carry):
        deg_v[pl.ds(i * 16, 16)] = jnp.zeros((16,), jnp.float32)
        return carry

    lax.fori_loop(0, N // 16, zero, 0, unroll=8)

    for st in range(EPT // DSTG):
        b0 = wid * EPT + st * DSTG
        pltpu.sync_copy(col_hbm.at[pl.ds(b0, DSTG)], cbuf)
        pltpu.sync_copy(ew_hbm.at[pl.ds(b0, DSTG)], wbuf)

        def body(g, carry):
            c = cbuf[pl.ds(g * 16, 16)]
            w = wbuf[pl.ds(g * 16, 16)]
            plsc.addupdate_scatter(deg_v, [c], w)
            return carry

        lax.fori_loop(0, DSTG // 16, body, 0, unroll=4)

    for kb in range(NB):
        pltpu.sync_copy(deg_v.at[pl.ds(kb * BN, BN)], pdeg_hbm.at[kb, wid])


_deg_kernel = functools.partial(
    pl.kernel,
    out_type=jax.ShapeDtypeStruct((NB, NW, BN), jnp.float32),
    mesh=_mesh,
    compiler_params=_sc_params,
    scratch_types=[
        pltpu.VMEM((DSTG,), jnp.int32),
        pltpu.VMEM((DSTG,), jnp.float32),
        pltpu.VMEM((N,), jnp.float32),
    ],
)(_deg_body)


def _scale(buf, ibuf, r):
    def ebody(e, carry):
        wi = plsc.load_gather(
            ibuf,
            [jnp.full((16,), r, jnp.int32), jnp.full((16,), 2, jnp.int32),
             jnp.full((16,), e, jnp.int32)])
        w = plsc.bitcast(wi, jnp.float32)
        for q in range(D // 16):
            sl = pl.ds(q * 16, 16)
            buf[e, sl] = buf[e, sl] * w
        return carry

    lax.fori_loop(0, CH, ebody, 0, unroll=5)


def _edge_body(y_hbm, pk_hbm, out_hbm,
               ibuf, gA, gB, gC, acc,
               gsemA, gsemB, gsemC, ssemA, ssemB, ssemC):
    cid = lax.axis_index("c")
    sid = lax.axis_index("s")
    wid = cid * NS + sid

    def zero(i, carry):
        gA[i // 8, pl.ds((i % 8) * 16, 16)] = jnp.zeros((16,), jnp.float32)
        return carry

    lax.fori_loop(0, ZR * 8, zero, 0, unroll=8)
    for k in range(RPS // ZR):
        pltpu.sync_copy(gA, acc.at[pl.ds(sid * RPS + k * ZR, ZR)])
    plsc.subcore_barrier()

    bufs = (gA, gB, gC)
    gsems = (gsemA, gsemB, gsemC)
    ssems = (ssemA, ssemB, ssemC)

    def issue_g(b, c):
        pltpu.async_copy(y_hbm.at[ibuf.at[c, 0]], bufs[b], gsems[b])

    def wait_g(b, c):
        pltpu.make_async_copy(
            y_hbm.at[ibuf.at[c, 0]], bufs[b], gsems[b]).wait()

    def issue_s(b, c):
        pltpu.async_copy(bufs[b], acc.at[ibuf.at[c, 1]], ssems[b], add=True)

    def wait_s(b, c):
        pltpu.make_async_copy(bufs[b], acc.at[ibuf.at[c, 1]], ssems[b]).wait()

    def stage(st, carry):
        r0 = wid * RPT + st * RSTAGE
        pltpu.sync_copy(pk_hbm.at[pl.ds(r0, RSTAGE)], ibuf)
        issue_g(0, 0)
        issue_g(1, 1)
        wait_g(0, 0); _scale(gA, ibuf, 0); issue_s(0, 0)
        issue_g(2, 2)
        wait_g(1, 1); _scale(gB, ibuf, 1); issue_s(1, 1)
        wait_s(0, 0); issue_g(0, 3)
        wait_g(2, 2); _scale(gC, ibuf, 2); issue_s(2, 2)
        wait_s(1, 1); issue_g(1, 4)
        wait_g(0, 3); _scale(gA, ibuf, 3); issue_s(0, 3)
        wait_s(2, 2); issue_g(2, 5)
        wait_g(1, 4); _scale(gB, ibuf, 4); issue_s(1, 4)
        wait_s(0, 3); issue_g(0, 6)
        wait_g(2, 5); _scale(gC, ibuf, 5); issue_s(2, 5)
        wait_s(1, 4); issue_g(1, 7)
        wait_g(0, 6); _scale(gA, ibuf, 6); issue_s(0, 6)
        wait_s(2, 5)
        wait_g(1, 7); _scale(gB, ibuf, 7); issue_s(1, 7)
        wait_s(0, 6)
        wait_s(1, 7)
        return carry

    lax.fori_loop(0, NSTAGE, stage, 0)

    plsc.subcore_barrier()
    for k in range(RPS // ZR):
        r0 = sid * RPS + k * ZR
        pltpu.sync_copy(acc.at[pl.ds(r0, ZR)], gA)
        pltpu.sync_copy(gA, out_hbm.at[cid, pl.ds(r0, ZR)])


_edge_kernel = functools.partial(
    pl.kernel,
    out_type=jax.ShapeDtypeStruct((NC, N, D), jnp.float32),
    mesh=_mesh,
    compiler_params=_sc_params,
    scratch_types=[
        pltpu.VMEM((RSTAGE, 3, CH), jnp.int32),
        pltpu.VMEM((CH, D), jnp.float32),
        pltpu.VMEM((CH, D), jnp.float32),
        pltpu.VMEM((CH, D), jnp.float32),
        pltpu.VMEM_SHARED((N, D), jnp.float32),
        pltpu.SemaphoreType.DMA,
        pltpu.SemaphoreType.DMA,
        pltpu.SemaphoreType.DMA,
        pltpu.SemaphoreType.DMA,
        pltpu.SemaphoreType.DMA,
        pltpu.SemaphoreType.DMA,
    ],
)(_edge_body)


def _dinv_block(pdeg_ref):
    deg = 1.0 + jnp.sum(pdeg_ref[0], axis=0)
    return lax.rsqrt(jnp.maximum(deg, 1e-12))


def _prep_body(x_ref, w_ref, pdeg_ref, y_ref):
    dinv = _dinv_block(pdeg_ref)
    xw = jnp.dot(x_ref[...], w_ref[...], preferred_element_type=jnp.float32)
    y_ref[...] = dinv[:, None] * xw


def _tc_prep(x, Wc1, pdeg):
    return pl.pallas_call(
        _prep_body,
        grid=(NB,),
        in_specs=[
            pl.BlockSpec((BN, D), lambda i: (i, 0)),
            pl.BlockSpec((D, D), lambda i: (0, 0)),
            pl.BlockSpec((1, NW, BN), lambda i: (i, 0, 0)),
        ],
        out_specs=pl.BlockSpec((BN, D), lambda i: (i, 0)),
        out_shape=jax.ShapeDtypeStruct((N, D), jnp.float32),
    )(x, Wc1, pdeg)


def _mid_body(acc_ref, y_ref, pdeg_ref, b_ref, w_ref, y2_ref):
    dinv = _dinv_block(pdeg_ref)
    t = acc_ref[0] + acc_ref[1] + y_ref[...]
    h = jnp.maximum(dinv[:, None] * t + b_ref[...], 0.0)
    hw = jnp.dot(h, w_ref[...], preferred_element_type=jnp.float32)
    y2_ref[...] = dinv[:, None] * hw


def _tc_mid(acc, y, pdeg, b, W):
    return pl.pallas_call(
        _mid_body,
        grid=(NB,),
        in_specs=[
            pl.BlockSpec((NC, BN, D), lambda i: (0, i, 0)),
            pl.BlockSpec((BN, D), lambda i: (i, 0)),
            pl.BlockSpec((1, NW, BN), lambda i: (i, 0, 0)),
            pl.BlockSpec((1, D), lambda i: (0, 0)),
            pl.BlockSpec((D, D), lambda i: (0, 0)),
        ],
        out_specs=pl.BlockSpec((BN, D), lambda i: (i, 0)),
        out_shape=jax.ShapeDtypeStruct((N, D), jnp.float32),
    )(acc, y, pdeg, b.reshape(1, D), W)


def _final_body(acc_ref, y_ref, pdeg_ref, b_ref, batch_ref,
                w0_ref, b0_ref, w1_ref, b1_ref, w2_ref, b2_ref,
                out_ref, sums, cnts):
    i = pl.program_id(0)
    dinv = _dinv_block(pdeg_ref)
    t = acc_ref[0] + acc_ref[1] + y_ref[...]
    h = jnp.maximum(dinv[:, None] * t + b_ref[...], 0.0)

    seg = lax.broadcasted_iota(jnp.int32, (G, BN), 0)
    oh = (batch_ref[0, 0, :][None, :] == seg).astype(jnp.float32)

    @pl.when(i == 0)
    def _():
        sums[...] = jnp.zeros((G, D), jnp.float32)
        cnts[...] = jnp.zeros((G, D), jnp.float32)

    sums[...] += jnp.dot(oh, h, preferred_element_type=jnp.float32)
    cnts[...] += jnp.dot(oh, jnp.ones((BN, D), jnp.float32),
                         preferred_element_type=jnp.float32)

    @pl.when(i == NB - 1)
    def _():
        g = sums[...] / jnp.maximum(cnts[...], 1.0)
        g = jnp.maximum(
            jnp.dot(g, w0_ref[...], preferred_element_type=jnp.float32)
            + b0_ref[...], 0.0)
        g = jnp.maximum(
            jnp.dot(g, w1_ref[...], preferred_element_type=jnp.float32)
            + b1_ref[...], 0.0)
        out_ref[...] = (
            jnp.dot(g, w2_ref[...], preferred_element_type=jnp.float32)
            + b2_ref[...])


def _tc_final(acc, y, pdeg, b, batch3, Wl0, bl0, Wl1, bl1, Wl2, bl2):
    return pl.pallas_call(
        _final_body,
        grid=(NB,),
        in_specs=[
            pl.BlockSpec((NC, BN, D), lambda i: (0, i, 0)),
            pl.BlockSpec((BN, D), lambda i: (i, 0)),
            pl.BlockSpec((1, NW, BN), lambda i: (i, 0, 0)),
            pl.BlockSpec((1, D), lambda i: (0, 0)),
            pl.BlockSpec((1, 1, BN), lambda i: (i, 0, 0)),
            pl.BlockSpec((D, D), lambda i: (0, 0)),
            pl.BlockSpec((1, D), lambda i: (0, 0)),
            pl.BlockSpec((D, D), lambda i: (0, 0)),
            pl.BlockSpec((1, D), lambda i: (0, 0)),
            pl.BlockSpec((D, OUTD), lambda i: (0, 0)),
            pl.BlockSpec((1, OUTD), lambda i: (0, 0)),
        ],
        out_specs=pl.BlockSpec((G, OUTD), lambda i: (0, 0)),
        out_shape=jax.ShapeDtypeStruct((G, OUTD), jnp.float32),
        scratch_shapes=[
            pltpu.VMEM((G, D), jnp.float32),
            pltpu.VMEM((G, D), jnp.float32),
        ],
    )(acc, y, pdeg, b.reshape(1, D), batch3,
      Wl0, bl0.reshape(1, D), Wl1, bl1.reshape(1, D),
      Wl2, bl2.reshape(1, OUTD))


@jax.jit
def kernel(x, edge_attr, edge_index, batch,
           Wc1, bc1, Wc2, bc2, Wl0, bl0, Wl1, bl1, Wl2, bl2):
    row = edge_index[0].astype(jnp.int32)
    col = edge_index[1].astype(jnp.int32)
    ewi = lax.bitcast_convert_type(edge_attr, jnp.int32)
    pk = jnp.stack([row.reshape(NROW, CH), col.reshape(NROW, CH),
                    ewi.reshape(NROW, CH)], axis=1)
    batch3 = batch.astype(jnp.int32).reshape(NB, 1, BN)

    pdeg = _deg_kernel(col, edge_attr)
    y1 = _tc_prep(x, Wc1, pdeg)
    acc1 = _edge_kernel(y1, pk)
    y2 = _tc_mid(acc1, y1, pdeg, bc1, Wc2)
    acc2 = _edge_kernel(y2, pk)
    return _tc_final(acc2, y2, pdeg, bc2, batch3,
                     Wl0, bl0, Wl1, bl1, Wl2, bl2)
